# Initial kernel scaffold; baseline (speedup 1.0000x reference)
import jax
import jax.numpy as jnp
from jax.experimental import pallas as pl


def kernel(z_e, codebook):
    raise NotImplementedError("write your pallas kernel here")


# speedup vs baseline: 1.3616x; 1.3616x over previous
import jax
import jax.numpy as jnp
from jax.experimental import pallas as pl

_CODE_DIM = 32
_NUM_CODES = 8192
_TOKEN_BLOCK = 512


def _vq_kernel(x_ref, cb_ref, out_ref):
    x = x_ref[...]
    cb = cb_ref[...]
    xn = x / jnp.maximum(
        jnp.sqrt(jnp.sum(x * x, axis=1, keepdims=True)), 1e-8)
    cbn = cb / jnp.maximum(
        jnp.sqrt(jnp.sum(cb * cb, axis=1, keepdims=True)), 1e-8)
    logits = jax.lax.dot_general(
        xn, cbn, (((1,), (1,)), ((), ())),
        preferred_element_type=jnp.float32)
    out_ref[0, 0, :] = jnp.argmax(logits, axis=1).astype(jnp.int32)


def kernel(z_e, codebook):
    b, t, d = z_e.shape
    n_tokens = b * t
    flat = z_e.reshape(n_tokens, d)
    n_blocks = n_tokens // _TOKEN_BLOCK

    out = pl.pallas_call(
        _vq_kernel,
        grid=(n_blocks,),
        in_specs=[
            pl.BlockSpec((_TOKEN_BLOCK, _CODE_DIM), lambda i: (i, 0)),
            pl.BlockSpec((_NUM_CODES, _CODE_DIM), lambda i: (0, 0)),
        ],
        out_specs=pl.BlockSpec((1, 1, _TOKEN_BLOCK), lambda i: (i, 0, 0)),
        out_shape=jax.ShapeDtypeStruct((n_blocks, 1, _TOKEN_BLOCK), jnp.int32),
    )(flat, codebook)
    return out.reshape(b, t)


# speedup vs baseline: 1.7435x; 1.2805x over previous
import jax
import jax.numpy as jnp
from jax.experimental import pallas as pl

_CODE_DIM = 32
_NUM_CODES = 8192
_TOKEN_BLOCK = 512


def _normalize_cb_kernel(cb_ref, out_ref):
    cb = cb_ref[...]
    out_ref[...] = cb / jnp.maximum(
        jnp.sqrt(jnp.sum(cb * cb, axis=1, keepdims=True)), 1e-8)


def _vq_kernel(x_ref, cbn_ref, out_ref):
    x = x_ref[...]
    cbn = cbn_ref[...]
    xn = x / jnp.maximum(
        jnp.sqrt(jnp.sum(x * x, axis=1, keepdims=True)), 1e-8)
    logits = jax.lax.dot_general(
        xn, cbn, (((1,), (1,)), ((), ())),
        preferred_element_type=jnp.float32)
    out_ref[0, 0, :] = jnp.argmax(logits, axis=1).astype(jnp.int32)


def kernel(z_e, codebook):
    b, t, d = z_e.shape
    n_tokens = b * t
    flat = z_e.reshape(n_tokens, d)
    n_blocks = n_tokens // _TOKEN_BLOCK

    cbn = pl.pallas_call(
        _normalize_cb_kernel,
        out_shape=jax.ShapeDtypeStruct((_NUM_CODES, _CODE_DIM), jnp.float32),
    )(codebook)

    out = pl.pallas_call(
        _vq_kernel,
        grid=(n_blocks,),
        in_specs=[
            pl.BlockSpec((_TOKEN_BLOCK, _CODE_DIM), lambda i: (i, 0)),
            pl.BlockSpec((_NUM_CODES, _CODE_DIM), lambda i: (0, 0)),
        ],
        out_specs=pl.BlockSpec((1, 1, _TOKEN_BLOCK), lambda i: (i, 0, 0)),
        out_shape=jax.ShapeDtypeStruct((n_blocks, 1, _TOKEN_BLOCK), jnp.int32),
    )(flat, cbn)
    return out.reshape(b, t)
